# baseline (speedup 1.0000x reference)
import functools

import jax
import jax.numpy as jnp
from jax import lax
from jax.experimental import pallas as pl
from jax.experimental.pallas import tpu as pltpu
from jax.experimental.pallas import tpu_sc as plsc

N = 10000
E = 320000
D = 128
L = 5
BD = 15

NC = 2
NS = 16
NW = NC * NS
EPW = E // NW
CHUNK = 80
EPWP = 10240
NCHUNK = EPWP // CHUNK
PADW = EPWP - EPW
NPAD = 10240
RPS = NPAD // NS


def _sc_aggregate(h, src3, dst3):
    mesh = plsc.VectorSubcoreMesh(core_axis_name="c", subcore_axis_name="s")

    @functools.partial(
        pl.kernel,
        out_type=jax.ShapeDtypeStruct((NC, NPAD, D), jnp.float32),
        mesh=mesh,
        scratch_types=[
            pltpu.VMEM((4, CHUNK), jnp.int32),
            pltpu.VMEM((4, CHUNK), jnp.int32),
            pltpu.VMEM((CHUNK, D), jnp.float32),
            pltpu.VMEM((CHUNK, D), jnp.float32),
            pltpu.VMEM((CHUNK, D), jnp.float32),
            pltpu.VMEM((CHUNK, D), jnp.float32),
            pltpu.VMEM_SHARED((NPAD, D), jnp.float32),
        ] + [pltpu.SemaphoreType.DMA] * 12,
    )
    def k(h_hbm, src_hbm, dst_hbm, out_hbm, sbuf, dbuf, r0, r1, r2, r3,
          acc, *sems):
        cid = lax.axis_index("c")
        sid = lax.axis_index("s")
        wid = cid * NS + sid
        rows = (r0, r1, r2, r3)
        sems_i = sems[0:4]
        sems_g = sems[4:8]
        sems_c = sems[8:12]

        @pl.loop(0, CHUNK)
        def _(i):
            @pl.loop(0, D, step=16)
            def _(j):
                r0[i, pl.ds(j, 16)] = jnp.zeros((16,), jnp.float32)

        @pl.loop(0, RPS, step=CHUNK)
        def _(r):
            pltpu.sync_copy(r0, acc.at[pl.ds(sid * RPS + r, CHUNK)])

        plsc.subcore_barrier()

        def idx_start(c, b):
            pltpu.make_async_copy(
                src_hbm.at[wid, c, :], sbuf.at[b], sems_i[b]).start()
            pltpu.make_async_copy(
                dst_hbm.at[wid, c, :], dbuf.at[b], sems_i[b]).start()

        def idx_wait(b):
            pltpu.make_async_copy(
                src_hbm.at[wid, 0, :], sbuf.at[b], sems_i[b]).wait()
            pltpu.make_async_copy(
                dst_hbm.at[wid, 0, :], dbuf.at[b], sems_i[b]).wait()

        def gather_start(b):
            pltpu.make_async_copy(
                h_hbm.at[sbuf.at[b]], rows[b], sems_g[b]).start()

        def gather_wait(b):
            pltpu.make_async_copy(
                h_hbm.at[sbuf.at[b]], rows[b], sems_g[b]).wait()

        def scat_start(b):
            pltpu.async_copy(rows[b], acc.at[dbuf.at[b]], sems_c[b],
                             add=True)

        def scat_wait(b):
            pltpu.make_async_copy(rows[b], acc.at[dbuf.at[b]],
                                  sems_c[b]).wait()

        idx_start(0, 0)
        idx_start(1, 1)
        idx_wait(0)
        gather_start(0)

        def step(c, b):
            b1 = (b + 1) % 4
            b2 = (b + 2) % 4

            @pl.when(c + 1 < NCHUNK)
            def _():
                idx_wait(b1)
                gather_start(b1)

            gather_wait(b)
            scat_start(b)

            @pl.when(c >= 2)
            def _():
                scat_wait(b2)

            @pl.when(c + 2 < NCHUNK)
            def _():
                idx_start(c + 2, b2)

        @pl.loop(0, NCHUNK, step=4)
        def _(i):
            step(i, 0)
            step(i + 1, 1)
            step(i + 2, 2)
            step(i + 3, 3)

        scat_wait((NCHUNK - 2) % 4)
        scat_wait((NCHUNK - 1) % 4)

        plsc.subcore_barrier()

        pltpu.sync_copy(
            acc.at[pl.ds(sid * RPS, RPS)],
            out_hbm.at[cid, pl.ds(sid * RPS, RPS)],
        )

    return k(h, src3, dst3)


def _tc_layer_body(h_ref, a0_ref, a1_ref, W_ref, b_ref, g_ref, be_ref,
                   pW10_ref, pb10_ref, pW20_ref, pb20_ref, pg0_ref, pbe0_ref,
                   pW11_ref, pb11_ref, pW21_ref, pb21_ref, pg1_ref, pbe1_ref,
                   gate_ref, out_ref, *, last):
    f32 = jnp.float32
    h = h_ref[...]
    aggr = a0_ref[...][:N] + a1_ref[...][:N]

    def bn(v, g, b):
        mu = jnp.mean(v, axis=0, keepdims=True)
        var = jnp.mean((v - mu) ** 2, axis=0, keepdims=True)
        return g * (v - mu) / jnp.sqrt(var + 1e-5) + b

    bf16 = jnp.bfloat16

    def dot(a, b):
        return lax.dot_general(a.astype(bf16), b.astype(bf16),
                               (((1,), (0,)), ((), ())),
                               preferred_element_type=f32)

    hm = dot(h + aggr, W_ref[...]) + b_ref[...]
    hb = bn(hm, g_ref[...], be_ref[...])

    def adapter(v, pW1, pb1, pW2, pb2, pg, pbe):
        t = jnp.maximum(dot(v, pW1) + pb1, 0.0)
        t = dot(t, pW2) + pb2
        return bn(t, pg, pbe)

    d0 = adapter(h, pW10_ref[...], pb10_ref[...], pW20_ref[...],
                 pb20_ref[...], pg0_ref[...], pbe0_ref[...])
    d1 = adapter(aggr, pW11_ref[...], pb11_ref[...], pW21_ref[...],
                 pb21_ref[...], pg1_ref[...], pbe1_ref[...])
    gate = gate_ref[...]
    hb = hb + d0 * gate[0] + d1 * gate[1]
    if not last:
        hb = jnp.maximum(hb, 0.0)
    out_ref[...] = hb


def _tc_layer(h, a0, a1, Wl, bl, gl, bel, p0, p1, gate, last):
    body = functools.partial(_tc_layer_body, last=last)
    return pl.pallas_call(
        body,
        out_shape=jax.ShapeDtypeStruct((N, D), jnp.float32),
        compiler_params=pltpu.CompilerParams(
            vmem_limit_bytes=56 * 2**20),
    )(h, a0, a1, Wl, bl, gl, bel, *p0, *p1, gate)


def kernel(x, edge_index, W_conv, b_conv, bn_gamma, bn_beta, pW1, pb1, pW2,
           pb2, pbn_gamma, pbn_beta, gating):
    srcw = edge_index[0].reshape(NW, EPW)
    dstw = edge_index[1].reshape(NW, EPW)
    dpad = jnp.broadcast_to(N + jnp.arange(PADW, dtype=jnp.int32), (NW, PADW))
    src3 = jnp.concatenate([srcw, srcw[:, :PADW]], axis=1)
    src3 = src3.reshape(NW, NCHUNK, CHUNK)
    dst3 = jnp.concatenate([dstw, dpad], axis=1).reshape(NW, NCHUNK, CHUNK)
    h = x
    for l in range(L):
        parts = _sc_aggregate(h, src3, dst3)
        p0 = (pW1[0, l], pb1[0, l], pW2[0, l], pb2[0, l],
              pbn_gamma[0, l], pbn_beta[0, l])
        p1 = (pW1[1, l], pb1[1, l], pW2[1, l], pb2[1, l],
              pbn_gamma[1, l], pbn_beta[1, l])
        gate = jnp.stack([gating[0, l, 0], gating[1, l, 0]])
        h = _tc_layer(h, parts[0], parts[1], W_conv[l], b_conv[l],
                      bn_gamma[l], bn_beta[l], p0, p1, gate, last=(l == L - 1))
    return h


# speedup vs baseline: 1.0330x; 1.0330x over previous
import functools

import jax
import jax.numpy as jnp
from jax import lax
from jax.experimental import pallas as pl
from jax.experimental.pallas import tpu as pltpu
from jax.experimental.pallas import tpu_sc as plsc

N = 10000
E = 320000
D = 128
L = 5
BD = 15

NC = 2
NS = 16
NW = NC * NS
EPW = E // NW
CHUNK = 128
EPWP = 10240
NCHUNK = EPWP // CHUNK
PADW = EPWP - EPW
NPAD = 10240
RPS = NPAD // NS


def _sc_aggregate(h, src3, dst3):
    mesh = plsc.VectorSubcoreMesh(core_axis_name="c", subcore_axis_name="s")

    @functools.partial(
        pl.kernel,
        out_type=jax.ShapeDtypeStruct((NC, NPAD, D), jnp.float32),
        mesh=mesh,
        scratch_types=[
            pltpu.VMEM((2, CHUNK), jnp.int32),
            pltpu.VMEM((2, CHUNK), jnp.int32),
            pltpu.VMEM((CHUNK, D), jnp.float32),
            pltpu.VMEM((CHUNK, D), jnp.float32),
            pltpu.VMEM_SHARED((NPAD, D), jnp.float32),
        ] + [pltpu.SemaphoreType.DMA] * 6,
    )
    def k(h_hbm, src_hbm, dst_hbm, out_hbm, sbuf, dbuf, rows0, rows1,
          acc, *sems):
        cid = lax.axis_index("c")
        sid = lax.axis_index("s")
        wid = cid * NS + sid
        rows = (rows0, rows1)
        sems_s = sems[0:2]
        sems_d = sems[2:4]
        sems_g = sems[4:6]

        @pl.loop(0, CHUNK)
        def _(i):
            @pl.loop(0, D, step=16)
            def _(j):
                rows0[i, pl.ds(j, 16)] = jnp.zeros((16,), jnp.float32)

        @pl.loop(0, RPS, step=CHUNK)
        def _(r):
            pltpu.sync_copy(rows0, acc.at[pl.ds(sid * RPS + r, CHUNK)])

        plsc.subcore_barrier()

        def idx_start(c, b):
            pltpu.make_async_copy(
                src_hbm.at[wid, c, :], sbuf.at[b], sems_s[b]).start()
            pltpu.make_async_copy(
                dst_hbm.at[wid, c, :], dbuf.at[b], sems_d[b]).start()

        def idx_wait(b):
            pltpu.make_async_copy(
                src_hbm.at[wid, 0, :], sbuf.at[b], sems_s[b]).wait()
            pltpu.make_async_copy(
                dst_hbm.at[wid, 0, :], dbuf.at[b], sems_d[b]).wait()

        def gather_start(b):
            pltpu.make_async_copy(
                h_hbm.at[sbuf.at[b]], rows[b], sems_g[b]).start()

        def gather_wait(b):
            pltpu.make_async_copy(
                h_hbm.at[sbuf.at[b]], rows[b], sems_g[b]).wait()

        def scat(b):
            pltpu.sync_copy(rows[b], acc.at[dbuf.at[b]], add=True)

        idx_start(0, 0)
        idx_start(1, 1)
        idx_wait(0)
        gather_start(0)

        def step(c, b, bn):
            @pl.when(c + 1 < NCHUNK)
            def _():
                idx_wait(bn)
                gather_start(bn)

            gather_wait(b)
            scat(b)

            @pl.when(c + 2 < NCHUNK)
            def _():
                idx_start(c + 2, b)

        @pl.loop(0, NCHUNK, step=2)
        def _(i):
            step(i, 0, 1)
            step(i + 1, 1, 0)

        plsc.subcore_barrier()

        pltpu.sync_copy(
            acc.at[pl.ds(sid * RPS, RPS)],
            out_hbm.at[cid, pl.ds(sid * RPS, RPS)],
        )

    return k(h, src3, dst3)


def _tc_layer_body(h_ref, a0_ref, a1_ref, W_ref, b_ref, g_ref, be_ref,
                   pW10_ref, pb10_ref, pW20_ref, pb20_ref, pg0_ref, pbe0_ref,
                   pW11_ref, pb11_ref, pW21_ref, pb21_ref, pg1_ref, pbe1_ref,
                   gate_ref, out_ref, *, last):
    f32 = jnp.float32
    h = h_ref[...]
    aggr = a0_ref[...][:N] + a1_ref[...][:N]

    def bn(v, g, b):
        mu = jnp.mean(v, axis=0, keepdims=True)
        var = jnp.mean((v - mu) ** 2, axis=0, keepdims=True)
        return g * (v - mu) / jnp.sqrt(var + 1e-5) + b

    bf16 = jnp.bfloat16

    def dot(a, b):
        return lax.dot_general(a.astype(bf16), b.astype(bf16),
                               (((1,), (0,)), ((), ())),
                               preferred_element_type=f32)

    hm = dot(h + aggr, W_ref[...]) + b_ref[...]
    hb = bn(hm, g_ref[...], be_ref[...])

    def adapter(v, pW1, pb1, pW2, pb2, pg, pbe):
        t = jnp.maximum(dot(v, pW1) + pb1, 0.0)
        t = dot(t, pW2) + pb2
        return bn(t, pg, pbe)

    d0 = adapter(h, pW10_ref[...], pb10_ref[...], pW20_ref[...],
                 pb20_ref[...], pg0_ref[...], pbe0_ref[...])
    d1 = adapter(aggr, pW11_ref[...], pb11_ref[...], pW21_ref[...],
                 pb21_ref[...], pg1_ref[...], pbe1_ref[...])
    gate = gate_ref[...]
    hb = hb + d0 * gate[0] + d1 * gate[1]
    if not last:
        hb = jnp.maximum(hb, 0.0)
    out_ref[...] = hb


def _tc_layer(h, a0, a1, Wl, bl, gl, bel, p0, p1, gate, last):
    body = functools.partial(_tc_layer_body, last=last)
    return pl.pallas_call(
        body,
        out_shape=jax.ShapeDtypeStruct((N, D), jnp.float32),
        compiler_params=pltpu.CompilerParams(
            vmem_limit_bytes=56 * 2**20),
    )(h, a0, a1, Wl, bl, gl, bel, *p0, *p1, gate)


def kernel(x, edge_index, W_conv, b_conv, bn_gamma, bn_beta, pW1, pb1, pW2,
           pb2, pbn_gamma, pbn_beta, gating):
    srcw = edge_index[0].reshape(NW, EPW)
    dstw = edge_index[1].reshape(NW, EPW)
    dpad = jnp.broadcast_to(N + jnp.arange(PADW, dtype=jnp.int32), (NW, PADW))
    src3 = jnp.concatenate([srcw, srcw[:, :PADW]], axis=1)
    src3 = src3.reshape(NW, NCHUNK, CHUNK)
    dst3 = jnp.concatenate([dstw, dpad], axis=1).reshape(NW, NCHUNK, CHUNK)
    h = x
    for l in range(L):
        parts = _sc_aggregate(h, src3, dst3)
        p0 = (pW1[0, l], pb1[0, l], pW2[0, l], pb2[0, l],
              pbn_gamma[0, l], pbn_beta[0, l])
        p1 = (pW1[1, l], pb1[1, l], pW2[1, l], pb2[1, l],
              pbn_gamma[1, l], pbn_beta[1, l])
        gate = jnp.stack([gating[0, l, 0], gating[1, l, 0]])
        h = _tc_layer(h, parts[0], parts[1], W_conv[l], b_conv[l],
                      bn_gamma[l], bn_beta[l], p0, p1, gate, last=(l == L - 1))
    return h
